# baseline (speedup 1.0000x reference)
import functools

import jax
import jax.numpy as jnp
from jax import lax
from jax.experimental import pallas as pl
from jax.experimental.pallas import tpu as pltpu
from jax.experimental.pallas import tpu_sc as plsc

DIM = 768
NUM_EXPERTS = 64
N_TOKENS = 32768
TOPK = 512
BT = 2048

L = 16
NV = N_TOKENS // L


def _gate_body(x_ref, w_ref, b_ref, out_ref):
    xb = x_ref[...]
    w = w_ref[...]
    logits = lax.dot_general(
        w, xb, (((1,), (1,)), ((), ())),
        preferred_element_type=jnp.float32)
    logits = logits + b_ref[...][:, None]
    scores = jax.nn.sigmoid(logits)
    out_ref[...] = lax.bitcast_convert_type(scores, jnp.int32)


def _gate_scores(x, W_gate, b_gate):
    grid = (N_TOKENS // BT,)
    return pl.pallas_call(
        _gate_body,
        grid=grid,
        in_specs=[
            pl.BlockSpec((BT, DIM), lambda i: (i, 0)),
            pl.BlockSpec((NUM_EXPERTS, DIM), lambda i: (0, 0)),
            pl.BlockSpec((NUM_EXPERTS,), lambda i: (0,)),
        ],
        out_specs=pl.BlockSpec((NUM_EXPERTS, BT), lambda i: (0, i)),
        out_shape=jax.ShapeDtypeStruct((NUM_EXPERTS, N_TOKENS), jnp.int32),
    )(x, W_gate, b_gate)


def _iota16():
    return lax.broadcasted_iota(jnp.int32, (L,), 0)


def _lane_cross(v, carry, target, iota):
    rv = lax.rev(v, (0,))
    dcum = lax.rev(plsc.cumsum(rv), (0,)) + carry
    cond_v = (dcum >= target).astype(jnp.int32)
    lane = jnp.sum(cond_v) - 1
    sel = iota == lane
    zeros = jnp.zeros((L,), jnp.int32)
    above = jnp.sum(jnp.where(sel, dcum - v, zeros))
    return lane, above


def _find_threshold(hist, coarse, ncoarse_v, target, smem, slot):
    iota = _iota16()

    def cond(state):
        _, carry = state
        return carry < target

    def body(state):
        j, carry = state
        v = coarse[pl.ds(j * L, L)]
        s = jnp.sum(v)
        new = carry + s

        @pl.when(new >= target)
        def _():
            lane, above = _lane_cross(v, carry, target, iota)
            smem[6] = j * L + lane
            smem[7] = above

        return j - 1, new

    lax.while_loop(cond, body, (jnp.int32(ncoarse_v - 1), jnp.int32(0)))

    cb = smem[6]
    carry2 = smem[7]
    v = hist[pl.ds(cb * L, L)]
    lane, above = _lane_cross(v, carry2, target, iota)
    smem[slot] = cb * L + lane
    smem[slot + 1] = above


def _topk_row(scores_hbm, vals_hbm, idx_hbm, e,
              keys, hist, coarse, selk, seli, selk2, seli2, bins, outv, smem):
    iota = _iota16()
    zeros = jnp.zeros((L,), jnp.int32)
    ones = jnp.ones((L,), jnp.int32)

    pltpu.sync_copy(scores_hbm.at[pl.ds(e * N_TOKENS, N_TOKENS)], keys)

    U = 8
    NCV = NV // L

    def clear_body(i, _):
        for u in range(U):
            hist[pl.ds((i * U + u) * L, L)] = zeros
        return 0

    def clear_coarse_body(i, _):
        for u in range(U):
            coarse[pl.ds((i * U + u) * L, L)] = zeros
        return 0

    lax.fori_loop(0, NV // U, clear_body, 0)
    lax.fori_loop(0, NCV // U, clear_coarse_body, 0)

    def hist_hi_body(i, _):
        ks = [keys[pl.ds((i * U + u) * L, L)] for u in range(U)]
        bs = [k >> 15 for k in ks]
        cbs = [k >> 19 for k in ks]
        for b, cb in zip(bs, cbs):
            plsc.addupdate_scatter(hist, [b], ones)
            plsc.addupdate_scatter(coarse, [cb], ones)
        return 0

    lax.fori_loop(0, NV // U, hist_hi_body, 0)

    _find_threshold(hist, coarse, NCV, jnp.int32(TOPK), smem, 0)
    h_star = smem[0]
    c_gt = smem[1]

    lax.fori_loop(0, NV // U, clear_body, 0)
    lax.fori_loop(0, NCV // U, clear_coarse_body, 0)

    def hist_lo_body(i, _):
        ks = [keys[pl.ds((i * U + u) * L, L)] for u in range(U)]
        els = [(k >> 15) == h_star for k in ks]
        lows = [k & 0x7FFF for k in ks]
        for lo, el in zip(lows, els):
            plsc.addupdate_scatter(hist, [lo], ones, mask=el)
            plsc.addupdate_scatter(coarse, [lo >> 4], ones, mask=el)
        return 0

    lax.fori_loop(0, NV // U, hist_lo_body, 0)

    _find_threshold(hist, coarse, NCV, TOPK - c_gt, smem, 3)
    l_star = smem[3]
    c_gt2 = smem[4]

    t_key = (h_star << 15) | l_star
    c_sel = c_gt + c_gt2

    UC = 8

    def gt_body(i, carry):
        offg, idxv = carry
        ks = [keys[pl.ds((i * UC + u) * L, L)] for u in range(UC)]
        gts = [k > t_key for k in ks]
        prefs = [plsc.cumsum(gt.astype(jnp.int32)) for gt in gts]
        cnts = [plsc.all_reduce_population_count(gt) for gt in gts]
        for u in range(UC):
            dest = offg + prefs[u] - 1
            plsc.store_scatter(selk, [dest], ks[u], mask=gts[u])
            plsc.store_scatter(seli, [dest], idxv + u * L, mask=gts[u])
            offg = offg + cnts[u]
        return offg, idxv + UC * L

    lax.fori_loop(0, NV // UC, gt_body, (zeros, iota))

    t_vec = zeros + t_key

    def tie_body(i, carry):
        offe, idxv = carry
        ks = [keys[pl.ds((i * UC + u) * L, L)] for u in range(UC)]
        eqs = [k == t_key for k in ks]
        prefs = [plsc.cumsum(eq.astype(jnp.int32)) for eq in eqs]
        cnts = [plsc.all_reduce_population_count(eq) for eq in eqs]
        for u in range(UC):
            dest = offe + prefs[u] - 1
            okm = jnp.logical_and(eqs[u], dest < TOPK)
            plsc.store_scatter(selk, [dest], t_vec, mask=okm)
            plsc.store_scatter(seli, [dest], idxv + u * L, mask=okm)
            offe = offe + cnts[u]
        return offe, idxv + UC * L

    lax.fori_loop(0, NV // UC, tie_body, (zeros + c_sel, iota))

    nv_sel = TOPK // L
    bufs = [(selk, seli), (selk2, seli2)]
    for p in range(6):
        srck, srci = bufs[p % 2]
        dstk, dsti = bufs[(p + 1) % 2]
        shift = 5 * p

        bins[pl.ds(0, L)] = zeros
        bins[pl.ds(L, L)] = zeros

        UB = 8

        def count_body(i, _, srck=srck, shift=shift):
            ks = [srck[pl.ds((i * UB + u) * L, L)] for u in range(UB)]
            dds = [31 - ((k >> shift) & 31) for k in ks]
            for dd in dds:
                plsc.addupdate_scatter(bins, [dd], ones)
            return 0

        lax.fori_loop(0, nv_sel // UB, count_body, 0)

        v0 = bins[pl.ds(0, L)]
        v1 = bins[pl.ds(L, L)]
        bins[pl.ds(0, L)] = plsc.cumsum(v0) - v0
        bins[pl.ds(L, L)] = plsc.cumsum(v1) - v1 + jnp.sum(v0)

        UP = 8

        def perm_body(i, _, srck=srck, srci=srci, dstk=dstk, dsti=dsti,
                      shift=shift):
            ks = [srck[pl.ds((i * UP + u) * L, L)] for u in range(UP)]
            ivs = [srci[pl.ds((i * UP + u) * L, L)] for u in range(UP)]
            dds = [31 - ((k >> shift) & 31) for k in ks]
            scans = [plsc.scan_count(dd) for dd in dds]
            for u in range(UP):
                occ, lm = scans[u]
                base = plsc.load_gather(bins, [dds[u]])
                dest = base + occ - 1
                plsc.store_scatter(dstk, [dest], ks[u])
                plsc.store_scatter(dsti, [dest], ivs[u])
                plsc.addupdate_scatter(bins, [dds[u]], occ, mask=lm)
            return 0

        lax.fori_loop(0, nv_sel // UP, perm_body, 0)

    UO = 8

    def out_body(i, _):
        ks = [selk[pl.ds((i * UO + u) * L, L)] for u in range(UO)]
        vs = [plsc.bitcast(k, jnp.float32) for k in ks]
        for u in range(UO):
            outv[pl.ds((i * UO + u) * L, L)] = vs[u]
        return 0

    lax.fori_loop(0, TOPK // L // UO, out_body, 0)

    pltpu.sync_copy(outv, vals_hbm.at[pl.ds(e * TOPK, TOPK)])
    pltpu.sync_copy(seli.at[pl.ds(0, TOPK)], idx_hbm.at[pl.ds(e * TOPK, TOPK)])


def _make_topk_sc():
    mesh = plsc.VectorSubcoreMesh(core_axis_name="c", subcore_axis_name="s")

    @functools.partial(
        pl.kernel,
        out_type=(
            jax.ShapeDtypeStruct((NUM_EXPERTS * TOPK,), jnp.float32),
            jax.ShapeDtypeStruct((NUM_EXPERTS * TOPK,), jnp.int32),
        ),
        mesh=mesh,
        compiler_params=pltpu.CompilerParams(needs_layout_passes=False),
        scratch_types=[
            pltpu.VMEM((N_TOKENS,), jnp.int32),
            pltpu.VMEM((N_TOKENS,), jnp.int32),
            pltpu.VMEM((N_TOKENS // L,), jnp.int32),
            pltpu.VMEM((TOPK,), jnp.int32),
            pltpu.VMEM((TOPK,), jnp.int32),
            pltpu.VMEM((TOPK,), jnp.int32),
            pltpu.VMEM((TOPK,), jnp.int32),
            pltpu.VMEM((2 * L,), jnp.int32),
            pltpu.VMEM((TOPK,), jnp.float32),
            pltpu.SMEM((8,), jnp.int32),
        ],
    )
    def topk_sc(scores_hbm, vals_hbm, idx_hbm,
                keys, hist, coarse, selk, seli, selk2, seli2, bins, outv,
                smem):
        wid = lax.axis_index("s") * 2 + lax.axis_index("c")
        for r in range(2):
            _topk_row(scores_hbm, vals_hbm, idx_hbm, wid * 2 + r,
                      keys, hist, coarse, selk, seli, selk2, seli2, bins,
                      outv, smem)

    return topk_sc


_topk_sc = _make_topk_sc()


@jax.jit
def kernel(x, W_gate, b_gate):
    score_bits = _gate_scores(x, W_gate, b_gate)
    vals, idx = _topk_sc(score_bits.reshape(-1))
    return vals.reshape(NUM_EXPERTS, TOPK), idx.reshape(NUM_EXPERTS, TOPK)


# speedup vs baseline: 1.0151x; 1.0151x over previous
import functools

import jax
import jax.numpy as jnp
from jax import lax
from jax.experimental import pallas as pl
from jax.experimental.pallas import tpu as pltpu
from jax.experimental.pallas import tpu_sc as plsc

DIM = 768
NUM_EXPERTS = 64
N_TOKENS = 32768
TOPK = 512
BT = 4096

L = 16
NV = N_TOKENS // L


def _gate_body(x_ref, w_ref, b_ref, out_ref):
    xb = x_ref[...]
    w = w_ref[...]
    logits = lax.dot_general(
        w, xb, (((1,), (1,)), ((), ())),
        preferred_element_type=jnp.float32)
    logits = logits + b_ref[...][:, None]
    scores = jax.nn.sigmoid(logits)
    out_ref[...] = lax.bitcast_convert_type(scores, jnp.int32)


def _gate_scores(x, W_gate, b_gate):
    grid = (N_TOKENS // BT,)
    return pl.pallas_call(
        _gate_body,
        grid=grid,
        in_specs=[
            pl.BlockSpec((BT, DIM), lambda i: (i, 0)),
            pl.BlockSpec((NUM_EXPERTS, DIM), lambda i: (0, 0)),
            pl.BlockSpec((NUM_EXPERTS,), lambda i: (0,)),
        ],
        out_specs=pl.BlockSpec((NUM_EXPERTS, BT), lambda i: (0, i)),
        out_shape=jax.ShapeDtypeStruct((NUM_EXPERTS, N_TOKENS), jnp.int32),
    )(x, W_gate, b_gate)


def _iota16():
    return lax.broadcasted_iota(jnp.int32, (L,), 0)


def _lane_cross(v, carry, target, iota):
    rv = lax.rev(v, (0,))
    dcum = lax.rev(plsc.cumsum(rv), (0,)) + carry
    cond_v = (dcum >= target).astype(jnp.int32)
    lane = jnp.sum(cond_v) - 1
    sel = iota == lane
    zeros = jnp.zeros((L,), jnp.int32)
    above = jnp.sum(jnp.where(sel, dcum - v, zeros))
    return lane, above


def _find_threshold(hist, coarse, ncoarse_v, target, smem, slot):
    iota = _iota16()

    def cond(state):
        _, carry = state
        return carry < target

    def body(state):
        j, carry = state
        v = coarse[pl.ds(j * L, L)]
        s = jnp.sum(v)
        new = carry + s

        @pl.when(new >= target)
        def _():
            lane, above = _lane_cross(v, carry, target, iota)
            smem[6] = j * L + lane
            smem[7] = above

        return j - 1, new

    lax.while_loop(cond, body, (jnp.int32(ncoarse_v - 1), jnp.int32(0)))

    cb = smem[6]
    carry2 = smem[7]
    v = hist[pl.ds(cb * L, L)]
    lane, above = _lane_cross(v, carry2, target, iota)
    smem[slot] = cb * L + lane
    smem[slot + 1] = above


def _topk_row(scores_hbm, vals_hbm, idx_hbm, e,
              keys, hist, coarse, selk, seli, selk2, seli2, bins, outv, smem):
    iota = _iota16()
    zeros = jnp.zeros((L,), jnp.int32)
    ones = jnp.ones((L,), jnp.int32)

    pltpu.sync_copy(scores_hbm.at[pl.ds(e * N_TOKENS, N_TOKENS)], keys)

    U = 8
    NCV = NV // L

    def clear_body(i, _):
        for u in range(U):
            hist[pl.ds((i * U + u) * L, L)] = zeros
        return 0

    def clear_coarse_body(i, _):
        for u in range(U):
            coarse[pl.ds((i * U + u) * L, L)] = zeros
        return 0

    lax.fori_loop(0, NV // U, clear_body, 0)
    lax.fori_loop(0, NCV // U, clear_coarse_body, 0)

    def hist_hi_body(i, _):
        ks = [keys[pl.ds((i * U + u) * L, L)] for u in range(U)]
        bs = [k >> 15 for k in ks]
        cbs = [k >> 19 for k in ks]
        for b, cb in zip(bs, cbs):
            plsc.addupdate_scatter(hist, [b], ones)
            plsc.addupdate_scatter(coarse, [cb], ones)
        return 0

    lax.fori_loop(0, NV // U, hist_hi_body, 0)

    _find_threshold(hist, coarse, NCV, jnp.int32(TOPK), smem, 0)
    h_star = smem[0]
    c_gt = smem[1]

    lax.fori_loop(0, NV // U, clear_body, 0)
    lax.fori_loop(0, NCV // U, clear_coarse_body, 0)

    def hist_lo_body(i, _):
        ks = [keys[pl.ds((i * U + u) * L, L)] for u in range(U)]
        els = [(k >> 15) == h_star for k in ks]
        lows = [k & 0x7FFF for k in ks]
        for lo, el in zip(lows, els):
            plsc.addupdate_scatter(hist, [lo], ones, mask=el)
            plsc.addupdate_scatter(coarse, [lo >> 4], ones, mask=el)
        return 0

    lax.fori_loop(0, NV // U, hist_lo_body, 0)

    _find_threshold(hist, coarse, NCV, TOPK - c_gt, smem, 3)
    l_star = smem[3]
    c_gt2 = smem[4]

    t_key = (h_star << 15) | l_star
    c_sel = c_gt + c_gt2

    UC = 8

    def gt_body(i, carry):
        offg, idxv = carry
        ks = [keys[pl.ds((i * UC + u) * L, L)] for u in range(UC)]
        gts = [k > t_key for k in ks]
        prefs = [plsc.cumsum(gt.astype(jnp.int32)) for gt in gts]
        cnts = [plsc.all_reduce_population_count(gt) for gt in gts]
        for u in range(UC):
            dest = offg + prefs[u] - 1
            plsc.store_scatter(selk, [dest], ks[u], mask=gts[u])
            plsc.store_scatter(seli, [dest], idxv + u * L, mask=gts[u])
            offg = offg + cnts[u]
        return offg, idxv + UC * L

    lax.fori_loop(0, NV // UC, gt_body, (zeros, iota))

    t_vec = zeros + t_key

    def tie_body(i, carry):
        offe, idxv = carry
        ks = [keys[pl.ds((i * UC + u) * L, L)] for u in range(UC)]
        eqs = [k == t_key for k in ks]
        prefs = [plsc.cumsum(eq.astype(jnp.int32)) for eq in eqs]
        cnts = [plsc.all_reduce_population_count(eq) for eq in eqs]
        for u in range(UC):
            dest = offe + prefs[u] - 1
            okm = jnp.logical_and(eqs[u], dest < TOPK)
            plsc.store_scatter(selk, [dest], t_vec, mask=okm)
            plsc.store_scatter(seli, [dest], idxv + u * L, mask=okm)
            offe = offe + cnts[u]
        return offe, idxv + UC * L

    lax.fori_loop(0, NV // UC, tie_body, (zeros + c_sel, iota))

    nv_sel = TOPK // L
    bufs = [(selk, seli), (selk2, seli2)]
    for p in range(6):
        srck, srci = bufs[p % 2]
        dstk, dsti = bufs[(p + 1) % 2]
        shift = 5 * p

        bins[pl.ds(0, L)] = zeros
        bins[pl.ds(L, L)] = zeros

        UB = 8

        def count_body(i, _, srck=srck, shift=shift):
            ks = [srck[pl.ds((i * UB + u) * L, L)] for u in range(UB)]
            dds = [31 - ((k >> shift) & 31) for k in ks]
            for dd in dds:
                plsc.addupdate_scatter(bins, [dd], ones)
            return 0

        lax.fori_loop(0, nv_sel // UB, count_body, 0)

        v0 = bins[pl.ds(0, L)]
        v1 = bins[pl.ds(L, L)]
        bins[pl.ds(0, L)] = plsc.cumsum(v0) - v0
        bins[pl.ds(L, L)] = plsc.cumsum(v1) - v1 + jnp.sum(v0)

        UP = 4

        def perm_body(i, _, srck=srck, srci=srci, dstk=dstk, dsti=dsti,
                      shift=shift):
            ks = [srck[pl.ds((i * UP + u) * L, L)] for u in range(UP)]
            ivs = [srci[pl.ds((i * UP + u) * L, L)] for u in range(UP)]
            dds = [31 - ((k >> shift) & 31) for k in ks]
            scans = [plsc.scan_count(dd) for dd in dds]
            for u in range(UP):
                occ, lm = scans[u]
                base = plsc.load_gather(bins, [dds[u]])
                dest = base + occ - 1
                plsc.store_scatter(dstk, [dest], ks[u])
                plsc.store_scatter(dsti, [dest], ivs[u])
                plsc.addupdate_scatter(bins, [dds[u]], occ, mask=lm)
            return 0

        lax.fori_loop(0, nv_sel // UP, perm_body, 0)

    UO = 8

    def out_body(i, _):
        ks = [selk[pl.ds((i * UO + u) * L, L)] for u in range(UO)]
        vs = [plsc.bitcast(k, jnp.float32) for k in ks]
        for u in range(UO):
            outv[pl.ds((i * UO + u) * L, L)] = vs[u]
        return 0

    lax.fori_loop(0, TOPK // L // UO, out_body, 0)

    pltpu.sync_copy(outv, vals_hbm.at[pl.ds(e * TOPK, TOPK)])
    pltpu.sync_copy(seli.at[pl.ds(0, TOPK)], idx_hbm.at[pl.ds(e * TOPK, TOPK)])


def _make_topk_sc():
    mesh = plsc.VectorSubcoreMesh(core_axis_name="c", subcore_axis_name="s")

    @functools.partial(
        pl.kernel,
        out_type=(
            jax.ShapeDtypeStruct((NUM_EXPERTS * TOPK,), jnp.float32),
            jax.ShapeDtypeStruct((NUM_EXPERTS * TOPK,), jnp.int32),
        ),
        mesh=mesh,
        compiler_params=pltpu.CompilerParams(needs_layout_passes=False),
        scratch_types=[
            pltpu.VMEM((N_TOKENS,), jnp.int32),
            pltpu.VMEM((N_TOKENS,), jnp.int32),
            pltpu.VMEM((N_TOKENS // L,), jnp.int32),
            pltpu.VMEM((TOPK,), jnp.int32),
            pltpu.VMEM((TOPK,), jnp.int32),
            pltpu.VMEM((TOPK,), jnp.int32),
            pltpu.VMEM((TOPK,), jnp.int32),
            pltpu.VMEM((2 * L,), jnp.int32),
            pltpu.VMEM((TOPK,), jnp.float32),
            pltpu.SMEM((8,), jnp.int32),
        ],
    )
    def topk_sc(scores_hbm, vals_hbm, idx_hbm,
                keys, hist, coarse, selk, seli, selk2, seli2, bins, outv,
                smem):
        wid = lax.axis_index("s") * 2 + lax.axis_index("c")
        for r in range(2):
            _topk_row(scores_hbm, vals_hbm, idx_hbm, wid * 2 + r,
                      keys, hist, coarse, selk, seli, selk2, seli2, bins,
                      outv, smem)

    return topk_sc


_topk_sc = _make_topk_sc()


@jax.jit
def kernel(x, W_gate, b_gate):
    score_bits = _gate_scores(x, W_gate, b_gate)
    vals, idx = _topk_sc(score_bits.reshape(-1))
    return vals.reshape(NUM_EXPERTS, TOPK), idx.reshape(NUM_EXPERTS, TOPK)


# speedup vs baseline: 1.0224x; 1.0072x over previous
import functools

import jax
import jax.numpy as jnp
from jax import lax
from jax.experimental import pallas as pl
from jax.experimental.pallas import tpu as pltpu
from jax.experimental.pallas import tpu_sc as plsc

DIM = 768
NUM_EXPERTS = 64
N_TOKENS = 32768
TOPK = 512
BT = 4096

L = 16
NV = N_TOKENS // L


def _gate_body(x_ref, w_ref, b_ref, out_ref):
    xb = x_ref[...]
    w = w_ref[...]
    logits = lax.dot_general(
        w, xb, (((1,), (1,)), ((), ())),
        preferred_element_type=jnp.float32)
    logits = logits + b_ref[...][:, None]
    scores = jax.nn.sigmoid(logits)
    out_ref[...] = lax.bitcast_convert_type(scores, jnp.int32)


def _gate_scores(x, W_gate, b_gate):
    grid = (N_TOKENS // BT,)
    return pl.pallas_call(
        _gate_body,
        grid=grid,
        in_specs=[
            pl.BlockSpec((BT, DIM), lambda i: (i, 0)),
            pl.BlockSpec((NUM_EXPERTS, DIM), lambda i: (0, 0)),
            pl.BlockSpec((NUM_EXPERTS,), lambda i: (0,)),
        ],
        out_specs=pl.BlockSpec((NUM_EXPERTS, BT), lambda i: (0, i)),
        out_shape=jax.ShapeDtypeStruct((NUM_EXPERTS, N_TOKENS), jnp.int32),
    )(x, W_gate, b_gate)


def _iota16():
    return lax.broadcasted_iota(jnp.int32, (L,), 0)


def _lane_cross(v, carry, target, iota):
    rv = lax.rev(v, (0,))
    dcum = lax.rev(plsc.cumsum(rv), (0,)) + carry
    cond_v = (dcum >= target).astype(jnp.int32)
    lane = jnp.sum(cond_v) - 1
    sel = iota == lane
    zeros = jnp.zeros((L,), jnp.int32)
    above = jnp.sum(jnp.where(sel, dcum - v, zeros))
    return lane, above


def _find_threshold(hist, coarse, ncoarse_v, target, smem, slot):
    iota = _iota16()

    def cond(state):
        _, carry = state
        return carry < target

    def body(state):
        j, carry = state
        v = coarse[pl.ds(j * L, L)]
        s = jnp.sum(v)
        new = carry + s

        @pl.when(new >= target)
        def _():
            lane, above = _lane_cross(v, carry, target, iota)
            smem[6] = j * L + lane
            smem[7] = above

        return j - 1, new

    lax.while_loop(cond, body, (jnp.int32(ncoarse_v - 1), jnp.int32(0)))

    cb = smem[6]
    carry2 = smem[7]
    v = hist[pl.ds(cb * L, L)]
    lane, above = _lane_cross(v, carry2, target, iota)
    smem[slot] = cb * L + lane
    smem[slot + 1] = above


def _topk_row(scores_hbm, vals_hbm, idx_hbm, e,
              keys, hist, coarse, selk, seli, selk2, seli2, bins, outv, smem):
    iota = _iota16()
    zeros = jnp.zeros((L,), jnp.int32)
    ones = jnp.ones((L,), jnp.int32)

    pltpu.sync_copy(scores_hbm.at[pl.ds(e * N_TOKENS, N_TOKENS)], keys)

    U = 8
    NCV = NV // L

    def clear_body(i, _):
        for u in range(U):
            hist[pl.ds((i * U + u) * L, L)] = zeros
        return 0

    def clear_coarse_body(i, _):
        for u in range(U):
            coarse[pl.ds((i * U + u) * L, L)] = zeros
        return 0

    lax.fori_loop(0, NV // U, clear_body, 0)
    lax.fori_loop(0, NCV // U, clear_coarse_body, 0)

    def hist_hi_body(i, _):
        ks = [keys[pl.ds((i * U + u) * L, L)] for u in range(U)]
        bs = [k >> 15 for k in ks]
        cbs = [k >> 19 for k in ks]
        for b, cb in zip(bs, cbs):
            plsc.addupdate_scatter(hist, [b], ones)
            plsc.addupdate_scatter(coarse, [cb], ones)
        return 0

    lax.fori_loop(0, NV // U, hist_hi_body, 0)

    _find_threshold(hist, coarse, NCV, jnp.int32(TOPK), smem, 0)
    h_star = smem[0]
    c_gt = smem[1]

    lax.fori_loop(0, NV // U, clear_body, 0)
    lax.fori_loop(0, NCV // U, clear_coarse_body, 0)

    def hist_lo_body(i, _):
        ks = [keys[pl.ds((i * U + u) * L, L)] for u in range(U)]
        els = [(k >> 15) == h_star for k in ks]
        lows = [k & 0x7FFF for k in ks]
        for lo, el in zip(lows, els):
            plsc.addupdate_scatter(hist, [lo], ones, mask=el)
            plsc.addupdate_scatter(coarse, [lo >> 4], ones, mask=el)
        return 0

    lax.fori_loop(0, NV // U, hist_lo_body, 0)

    _find_threshold(hist, coarse, NCV, TOPK - c_gt, smem, 3)
    l_star = smem[3]
    c_gt2 = smem[4]

    t_key = (h_star << 15) | l_star
    c_sel = c_gt + c_gt2

    UC = 8

    def gt_body(i, carry):
        offg, idxv = carry
        ks = [keys[pl.ds((i * UC + u) * L, L)] for u in range(UC)]
        gts = [k > t_key for k in ks]
        prefs = [plsc.cumsum(gt.astype(jnp.int32)) for gt in gts]
        cnts = [plsc.all_reduce_population_count(gt) for gt in gts]
        for u in range(UC):
            dest = offg + prefs[u] - 1
            plsc.store_scatter(selk, [dest], ks[u], mask=gts[u])
            plsc.store_scatter(seli, [dest], idxv + u * L, mask=gts[u])
            offg = offg + cnts[u]
        return offg, idxv + UC * L

    lax.fori_loop(0, NV // UC, gt_body, (zeros, iota))

    t_vec = zeros + t_key

    def tie_body(i, carry):
        offe, idxv = carry
        ks = [keys[pl.ds((i * UC + u) * L, L)] for u in range(UC)]
        eqs = [k == t_key for k in ks]
        prefs = [plsc.cumsum(eq.astype(jnp.int32)) for eq in eqs]
        cnts = [plsc.all_reduce_population_count(eq) for eq in eqs]
        for u in range(UC):
            dest = offe + prefs[u] - 1
            okm = jnp.logical_and(eqs[u], dest < TOPK)
            plsc.store_scatter(selk, [dest], t_vec, mask=okm)
            plsc.store_scatter(seli, [dest], idxv + u * L, mask=okm)
            offe = offe + cnts[u]
        return offe, idxv + UC * L

    lax.fori_loop(0, NV // UC, tie_body, (zeros + c_sel, iota))

    nv_sel = TOPK // L
    bufs = [(selk, seli), (selk2, seli2)]
    for p in range(6):
        srck, srci = bufs[p % 2]
        dstk, dsti = bufs[(p + 1) % 2]
        shift = 5 * p

        bins[pl.ds(0, L)] = zeros
        bins[pl.ds(L, L)] = zeros

        UB = 8

        def count_body(i, _, srck=srck, shift=shift):
            ks = [srck[pl.ds((i * UB + u) * L, L)] for u in range(UB)]
            dds = [31 - ((k >> shift) & 31) for k in ks]
            for dd in dds:
                plsc.addupdate_scatter(bins, [dd], ones)
            return 0

        lax.fori_loop(0, nv_sel // UB, count_body, 0)

        v0 = bins[pl.ds(0, L)]
        v1 = bins[pl.ds(L, L)]
        bins[pl.ds(0, L)] = plsc.cumsum(v0) - v0
        bins[pl.ds(L, L)] = plsc.cumsum(v1) - v1 + jnp.sum(v0)

        UP = 4

        def perm_body(i, _, srck=srck, srci=srci, dstk=dstk, dsti=dsti,
                      shift=shift):
            ks = [srck[pl.ds((i * UP + u) * L, L)] for u in range(UP)]
            ivs = [srci[pl.ds((i * UP + u) * L, L)] for u in range(UP)]
            dds = [31 - ((k >> shift) & 31) for k in ks]
            scans = [plsc.scan_count(dd) for dd in dds]
            for u in range(UP):
                occ, lm = scans[u]
                base = plsc.load_gather(bins, [dds[u]])
                plsc.addupdate_scatter(bins, [dds[u]], occ, mask=lm)
                dest = base + occ - 1
                plsc.store_scatter(dstk, [dest], ks[u])
                plsc.store_scatter(dsti, [dest], ivs[u])
            return 0

        lax.fori_loop(0, nv_sel // UP, perm_body, 0)

    UO = 8

    def out_body(i, _):
        ks = [selk[pl.ds((i * UO + u) * L, L)] for u in range(UO)]
        vs = [plsc.bitcast(k, jnp.float32) for k in ks]
        for u in range(UO):
            outv[pl.ds((i * UO + u) * L, L)] = vs[u]
        return 0

    lax.fori_loop(0, TOPK // L // UO, out_body, 0)

    pltpu.sync_copy(outv, vals_hbm.at[pl.ds(e * TOPK, TOPK)])
    pltpu.sync_copy(seli.at[pl.ds(0, TOPK)], idx_hbm.at[pl.ds(e * TOPK, TOPK)])


def _make_topk_sc():
    mesh = plsc.VectorSubcoreMesh(core_axis_name="c", subcore_axis_name="s")

    @functools.partial(
        pl.kernel,
        out_type=(
            jax.ShapeDtypeStruct((NUM_EXPERTS * TOPK,), jnp.float32),
            jax.ShapeDtypeStruct((NUM_EXPERTS * TOPK,), jnp.int32),
        ),
        mesh=mesh,
        compiler_params=pltpu.CompilerParams(needs_layout_passes=False),
        scratch_types=[
            pltpu.VMEM((N_TOKENS,), jnp.int32),
            pltpu.VMEM((N_TOKENS,), jnp.int32),
            pltpu.VMEM((N_TOKENS // L,), jnp.int32),
            pltpu.VMEM((TOPK,), jnp.int32),
            pltpu.VMEM((TOPK,), jnp.int32),
            pltpu.VMEM((TOPK,), jnp.int32),
            pltpu.VMEM((TOPK,), jnp.int32),
            pltpu.VMEM((2 * L,), jnp.int32),
            pltpu.VMEM((TOPK,), jnp.float32),
            pltpu.SMEM((8,), jnp.int32),
        ],
    )
    def topk_sc(scores_hbm, vals_hbm, idx_hbm,
                keys, hist, coarse, selk, seli, selk2, seli2, bins, outv,
                smem):
        wid = lax.axis_index("s") * 2 + lax.axis_index("c")
        for r in range(2):
            _topk_row(scores_hbm, vals_hbm, idx_hbm, wid * 2 + r,
                      keys, hist, coarse, selk, seli, selk2, seli2, bins,
                      outv, smem)

    return topk_sc


_topk_sc = _make_topk_sc()


@jax.jit
def kernel(x, W_gate, b_gate):
    score_bits = _gate_scores(x, W_gate, b_gate)
    vals, idx = _topk_sc(score_bits.reshape(-1))
    return vals.reshape(NUM_EXPERTS, TOPK), idx.reshape(NUM_EXPERTS, TOPK)


# speedup vs baseline: 1.1631x; 1.1376x over previous
import functools

import jax
import jax.numpy as jnp
from jax import lax
from jax.experimental import pallas as pl
from jax.experimental.pallas import tpu as pltpu
from jax.experimental.pallas import tpu_sc as plsc

DIM = 768
NUM_EXPERTS = 64
N_TOKENS = 32768
TOPK = 512
BT = 4096

L = 16
NV = N_TOKENS // L


def _gate_body(x_ref, w_ref, b_ref, out_ref):
    xb = x_ref[...]
    w = w_ref[...]
    logits = lax.dot_general(
        w, xb, (((1,), (1,)), ((), ())),
        preferred_element_type=jnp.float32)
    logits = logits + b_ref[...][:, None]
    scores = jax.nn.sigmoid(logits)
    bits = lax.bitcast_convert_type(scores, jnp.int32)
    out_ref[...] = bits.reshape(NUM_EXPERTS, BT // 1024, 8, 128)


def _gate_scores(x, W_gate, b_gate):
    grid = (N_TOKENS // BT,)
    return pl.pallas_call(
        _gate_body,
        grid=grid,
        in_specs=[
            pl.BlockSpec((BT, DIM), lambda i: (i, 0)),
            pl.BlockSpec((NUM_EXPERTS, DIM), lambda i: (0, 0)),
            pl.BlockSpec((NUM_EXPERTS,), lambda i: (0,)),
        ],
        out_specs=pl.BlockSpec(
            (NUM_EXPERTS, BT // 1024, 8, 128), lambda i: (0, i, 0, 0)),
        out_shape=jax.ShapeDtypeStruct(
            (NUM_EXPERTS, N_TOKENS // 1024, 8, 128), jnp.int32),
    )(x, W_gate, b_gate)


def _iota16():
    return lax.broadcasted_iota(jnp.int32, (L,), 0)


def _lane_cross(v, carry, target, iota):
    rv = lax.rev(v, (0,))
    dcum = lax.rev(plsc.cumsum(rv), (0,)) + carry
    cond_v = (dcum >= target).astype(jnp.int32)
    lane = jnp.sum(cond_v) - 1
    sel = iota == lane
    zeros = jnp.zeros((L,), jnp.int32)
    above = jnp.sum(jnp.where(sel, dcum - v, zeros))
    return lane, above


def _find_threshold(hist, coarse, ncoarse_v, target, smem, slot):
    iota = _iota16()

    def cond(state):
        _, carry = state
        return carry < target

    def body(state):
        j, carry = state
        v = coarse[pl.ds(j * L, L)]
        s = jnp.sum(v)
        new = carry + s

        @pl.when(new >= target)
        def _():
            lane, above = _lane_cross(v, carry, target, iota)
            smem[6] = j * L + lane
            smem[7] = above

        return j - 1, new

    lax.while_loop(cond, body, (jnp.int32(ncoarse_v - 1), jnp.int32(0)))

    cb = smem[6]
    carry2 = smem[7]
    v = hist[pl.ds(cb * L, L)]
    lane, above = _lane_cross(v, carry2, target, iota)
    smem[slot] = cb * L + lane
    smem[slot + 1] = above


def _topk_row(scores_hbm, vals_hbm, idx_hbm, e,
              keys, hist, coarse, selk, seli, selk2, seli2, bins, outv, smem):
    iota = _iota16()
    zeros = jnp.zeros((L,), jnp.int32)
    ones = jnp.ones((L,), jnp.int32)

    pltpu.sync_copy(scores_hbm.at[pl.ds(e * N_TOKENS, N_TOKENS)], keys)

    U = 8
    NCV = NV // L

    def clear_body(i, _):
        for u in range(U):
            hist[pl.ds((i * U + u) * L, L)] = zeros
        return 0

    def clear_coarse_body(i, _):
        for u in range(U):
            coarse[pl.ds((i * U + u) * L, L)] = zeros
        return 0

    lax.fori_loop(0, NV // U, clear_body, 0)
    lax.fori_loop(0, NCV // U, clear_coarse_body, 0)

    def hist_hi_body(i, _):
        ks = [keys[pl.ds((i * U + u) * L, L)] for u in range(U)]
        bs = [k >> 15 for k in ks]
        cbs = [k >> 19 for k in ks]
        for b, cb in zip(bs, cbs):
            plsc.addupdate_scatter(hist, [b], ones)
            plsc.addupdate_scatter(coarse, [cb], ones)
        return 0

    lax.fori_loop(0, NV // U, hist_hi_body, 0)

    _find_threshold(hist, coarse, NCV, jnp.int32(TOPK), smem, 0)
    h_star = smem[0]
    c_gt = smem[1]

    lax.fori_loop(0, NV // U, clear_body, 0)
    lax.fori_loop(0, NCV // U, clear_coarse_body, 0)

    def hist_lo_body(i, _):
        ks = [keys[pl.ds((i * U + u) * L, L)] for u in range(U)]
        els = [(k >> 15) == h_star for k in ks]
        lows = [k & 0x7FFF for k in ks]
        for lo, el in zip(lows, els):
            plsc.addupdate_scatter(hist, [lo], ones, mask=el)
            plsc.addupdate_scatter(coarse, [lo >> 4], ones, mask=el)
        return 0

    lax.fori_loop(0, NV // U, hist_lo_body, 0)

    _find_threshold(hist, coarse, NCV, TOPK - c_gt, smem, 3)
    l_star = smem[3]
    c_gt2 = smem[4]

    t_key = (h_star << 15) | l_star
    c_sel = c_gt + c_gt2

    UC = 8

    def gt_body(i, carry):
        offg, idxv = carry
        ks = [keys[pl.ds((i * UC + u) * L, L)] for u in range(UC)]
        gts = [k > t_key for k in ks]
        prefs = [plsc.cumsum(gt.astype(jnp.int32)) for gt in gts]
        cnts = [plsc.all_reduce_population_count(gt) for gt in gts]
        for u in range(UC):
            dest = offg + prefs[u] - 1
            plsc.store_scatter(selk, [dest], ks[u], mask=gts[u])
            plsc.store_scatter(seli, [dest], idxv + u * L, mask=gts[u])
            offg = offg + cnts[u]
        return offg, idxv + UC * L

    lax.fori_loop(0, NV // UC, gt_body, (zeros, iota))

    t_vec = zeros + t_key

    def tie_body(i, carry):
        offe, idxv = carry
        ks = [keys[pl.ds((i * UC + u) * L, L)] for u in range(UC)]
        eqs = [k == t_key for k in ks]
        prefs = [plsc.cumsum(eq.astype(jnp.int32)) for eq in eqs]
        cnts = [plsc.all_reduce_population_count(eq) for eq in eqs]
        for u in range(UC):
            dest = offe + prefs[u] - 1
            okm = jnp.logical_and(eqs[u], dest < TOPK)
            plsc.store_scatter(selk, [dest], t_vec, mask=okm)
            plsc.store_scatter(seli, [dest], idxv + u * L, mask=okm)
            offe = offe + cnts[u]
        return offe, idxv + UC * L

    lax.fori_loop(0, NV // UC, tie_body, (zeros + c_sel, iota))

    nv_sel = TOPK // L
    bufs = [(selk, seli), (selk2, seli2)]
    for p in range(6):
        srck, srci = bufs[p % 2]
        dstk, dsti = bufs[(p + 1) % 2]
        shift = 5 * p

        bins[pl.ds(0, L)] = zeros
        bins[pl.ds(L, L)] = zeros

        UB = 8

        def count_body(i, _, srck=srck, shift=shift):
            ks = [srck[pl.ds((i * UB + u) * L, L)] for u in range(UB)]
            dds = [31 - ((k >> shift) & 31) for k in ks]
            for dd in dds:
                plsc.addupdate_scatter(bins, [dd], ones)
            return 0

        lax.fori_loop(0, nv_sel // UB, count_body, 0)

        v0 = bins[pl.ds(0, L)]
        v1 = bins[pl.ds(L, L)]
        bins[pl.ds(0, L)] = plsc.cumsum(v0) - v0
        bins[pl.ds(L, L)] = plsc.cumsum(v1) - v1 + jnp.sum(v0)

        UP = 4

        def perm_body(i, _, srck=srck, srci=srci, dstk=dstk, dsti=dsti,
                      shift=shift):
            ks = [srck[pl.ds((i * UP + u) * L, L)] for u in range(UP)]
            ivs = [srci[pl.ds((i * UP + u) * L, L)] for u in range(UP)]
            dds = [31 - ((k >> shift) & 31) for k in ks]
            scans = [plsc.scan_count(dd) for dd in dds]
            for u in range(UP):
                occ, lm = scans[u]
                base = plsc.load_gather(bins, [dds[u]])
                plsc.addupdate_scatter(bins, [dds[u]], occ, mask=lm)
                dest = base + occ - 1
                plsc.store_scatter(dstk, [dest], ks[u])
                plsc.store_scatter(dsti, [dest], ivs[u])
            return 0

        lax.fori_loop(0, nv_sel // UP, perm_body, 0)

    UO = 8

    def out_body(i, _):
        ks = [selk[pl.ds((i * UO + u) * L, L)] for u in range(UO)]
        vs = [plsc.bitcast(k, jnp.float32) for k in ks]
        for u in range(UO):
            outv[pl.ds((i * UO + u) * L, L)] = vs[u]
        return 0

    lax.fori_loop(0, TOPK // L // UO, out_body, 0)

    pltpu.sync_copy(outv, vals_hbm.at[pl.ds(e * TOPK, TOPK)])
    pltpu.sync_copy(seli.at[pl.ds(0, TOPK)], idx_hbm.at[pl.ds(e * TOPK, TOPK)])


def _make_topk_sc():
    mesh = plsc.VectorSubcoreMesh(core_axis_name="c", subcore_axis_name="s")

    @functools.partial(
        pl.kernel,
        out_type=(
            jax.ShapeDtypeStruct((NUM_EXPERTS * TOPK,), jnp.float32),
            jax.ShapeDtypeStruct((NUM_EXPERTS * TOPK,), jnp.int32),
        ),
        mesh=mesh,
        compiler_params=pltpu.CompilerParams(needs_layout_passes=False),
        scratch_types=[
            pltpu.VMEM((N_TOKENS,), jnp.int32),
            pltpu.VMEM((N_TOKENS,), jnp.int32),
            pltpu.VMEM((N_TOKENS // L,), jnp.int32),
            pltpu.VMEM((TOPK,), jnp.int32),
            pltpu.VMEM((TOPK,), jnp.int32),
            pltpu.VMEM((TOPK,), jnp.int32),
            pltpu.VMEM((TOPK,), jnp.int32),
            pltpu.VMEM((2 * L,), jnp.int32),
            pltpu.VMEM((TOPK,), jnp.float32),
            pltpu.SMEM((8,), jnp.int32),
        ],
    )
    def topk_sc(scores_hbm, vals_hbm, idx_hbm,
                keys, hist, coarse, selk, seli, selk2, seli2, bins, outv,
                smem):
        wid = lax.axis_index("s") * 2 + lax.axis_index("c")
        for r in range(2):
            _topk_row(scores_hbm, vals_hbm, idx_hbm, wid * 2 + r,
                      keys, hist, coarse, selk, seli, selk2, seli2, bins,
                      outv, smem)

    return topk_sc


_topk_sc = _make_topk_sc()


@jax.jit
def kernel(x, W_gate, b_gate):
    score_bits = _gate_scores(x, W_gate, b_gate)
    vals, idx = _topk_sc(score_bits.reshape(-1))
    return vals.reshape(NUM_EXPERTS, TOPK), idx.reshape(NUM_EXPERTS, TOPK)


# speedup vs baseline: 1.2398x; 1.0659x over previous
import functools

import jax
import jax.numpy as jnp
from jax import lax
from jax.experimental import pallas as pl
from jax.experimental.pallas import tpu as pltpu
from jax.experimental.pallas import tpu_sc as plsc

DIM = 768
NUM_EXPERTS = 64
N_TOKENS = 32768
TOPK = 512
BT = 4096

L = 16
NV = N_TOKENS // L


def _gate_body(x_ref, w_ref, b_ref, out_ref):
    xb = x_ref[...]
    w = w_ref[...]
    logits = lax.dot_general(
        w, xb, (((1,), (1,)), ((), ())),
        preferred_element_type=jnp.float32)
    logits = logits + b_ref[...][:, None]
    scores = jax.nn.sigmoid(logits)
    bits = lax.bitcast_convert_type(scores, jnp.int32)
    out_ref[...] = bits.reshape(NUM_EXPERTS, BT // 1024, 8, 128)


def _gate_scores(x, W_gate, b_gate):
    grid = (N_TOKENS // BT,)
    return pl.pallas_call(
        _gate_body,
        grid=grid,
        in_specs=[
            pl.BlockSpec((BT, DIM), lambda i: (i, 0)),
            pl.BlockSpec((NUM_EXPERTS, DIM), lambda i: (0, 0)),
            pl.BlockSpec((NUM_EXPERTS,), lambda i: (0,)),
        ],
        out_specs=pl.BlockSpec(
            (NUM_EXPERTS, BT // 1024, 8, 128), lambda i: (0, i, 0, 0)),
        out_shape=jax.ShapeDtypeStruct(
            (NUM_EXPERTS, N_TOKENS // 1024, 8, 128), jnp.int32),
    )(x, W_gate, b_gate)


def _iota16():
    return lax.broadcasted_iota(jnp.int32, (L,), 0)


def _lane_cross(v, carry, target, iota):
    rv = lax.rev(v, (0,))
    dcum = lax.rev(plsc.cumsum(rv), (0,)) + carry
    cond_v = (dcum >= target).astype(jnp.int32)
    lane = jnp.sum(cond_v) - 1
    sel = iota == lane
    zeros = jnp.zeros((L,), jnp.int32)
    above = jnp.sum(jnp.where(sel, dcum - v, zeros))
    return lane, above


def _find_threshold(hist, coarse, ncoarse_v, target, smem, slot):
    iota = _iota16()

    def cond(state):
        _, carry = state
        return carry < target

    def body(state):
        j, carry = state
        v = coarse[pl.ds(j * L, L)]
        s = jnp.sum(v)
        new = carry + s

        @pl.when(new >= target)
        def _():
            lane, above = _lane_cross(v, carry, target, iota)
            smem[6] = j * L + lane
            smem[7] = above

        return j - 1, new

    lax.while_loop(cond, body, (jnp.int32(ncoarse_v - 1), jnp.int32(0)))

    cb = smem[6]
    carry2 = smem[7]
    v = hist[pl.ds(cb * L, L)]
    lane, above = _lane_cross(v, carry2, target, iota)
    smem[slot] = cb * L + lane
    smem[slot + 1] = above


def _topk_row(vals_hbm, idx_hbm, e,
              keys, hist, coarse, selk, seli, selk2, seli2, bins, outv, smem):
    iota = _iota16()
    zeros = jnp.zeros((L,), jnp.int32)
    ones = jnp.ones((L,), jnp.int32)

    U = 8
    NCV = NV // L

    def clear_body(i, _):
        for u in range(U):
            hist[pl.ds((i * U + u) * L, L)] = zeros
        return 0

    def clear_coarse_body(i, _):
        for u in range(U):
            coarse[pl.ds((i * U + u) * L, L)] = zeros
        return 0

    lax.fori_loop(0, NV // U, clear_body, 0)
    lax.fori_loop(0, NCV // U, clear_coarse_body, 0)

    def hist_hi_body(i, _):
        ks = [keys[pl.ds((i * U + u) * L, L)] for u in range(U)]
        bs = [k >> 15 for k in ks]
        cbs = [k >> 19 for k in ks]
        for b, cb in zip(bs, cbs):
            plsc.addupdate_scatter(hist, [b], ones)
            plsc.addupdate_scatter(coarse, [cb], ones)
        return 0

    lax.fori_loop(0, NV // U, hist_hi_body, 0)

    _find_threshold(hist, coarse, NCV, jnp.int32(TOPK), smem, 0)
    h_star = smem[0]
    c_gt = smem[1]

    lax.fori_loop(0, NV // U, clear_body, 0)
    lax.fori_loop(0, NCV // U, clear_coarse_body, 0)

    def hist_lo_body(i, _):
        ks = [keys[pl.ds((i * U + u) * L, L)] for u in range(U)]
        els = [(k >> 15) == h_star for k in ks]
        lows = [k & 0x7FFF for k in ks]
        for lo, el in zip(lows, els):
            plsc.addupdate_scatter(hist, [lo], ones, mask=el)
            plsc.addupdate_scatter(coarse, [lo >> 4], ones, mask=el)
        return 0

    lax.fori_loop(0, NV // U, hist_lo_body, 0)

    _find_threshold(hist, coarse, NCV, TOPK - c_gt, smem, 3)
    l_star = smem[3]
    c_gt2 = smem[4]

    t_key = (h_star << 15) | l_star
    c_sel = c_gt + c_gt2

    UC = 8
    t_vec = zeros + t_key

    def collect_body(i, carry):
        offg, offe, idxv = carry
        ks = [keys[pl.ds((i * UC + u) * L, L)] for u in range(UC)]
        gts = [k > t_key for k in ks]
        eqs = [k == t_key for k in ks]
        prefs_g = [plsc.cumsum(gt.astype(jnp.int32)) for gt in gts]
        prefs_e = [plsc.cumsum(eq.astype(jnp.int32)) for eq in eqs]
        cnts_g = [plsc.all_reduce_population_count(gt) for gt in gts]
        cnts_e = [plsc.all_reduce_population_count(eq) for eq in eqs]
        for u in range(UC):
            dest_g = offg + prefs_g[u] - 1
            plsc.store_scatter(selk, [dest_g], ks[u], mask=gts[u])
            plsc.store_scatter(seli, [dest_g], idxv + u * L, mask=gts[u])
            offg = offg + cnts_g[u]
        for u in range(UC):
            dest_e = offe + prefs_e[u] - 1
            okm = jnp.logical_and(eqs[u], dest_e < TOPK)
            plsc.store_scatter(selk, [dest_e], t_vec, mask=okm)
            plsc.store_scatter(seli, [dest_e], idxv + u * L, mask=okm)
            offe = offe + cnts_e[u]
        return offg, offe, idxv + UC * L

    lax.fori_loop(0, NV // UC, collect_body, (zeros, zeros + c_sel, iota))

    nv_sel = TOPK // L
    bufs = [(selk, seli), (selk2, seli2)]
    for p in range(6):
        srck, srci = bufs[p % 2]
        dstk, dsti = bufs[(p + 1) % 2]
        shift = 5 * p

        bins[pl.ds(0, L)] = zeros
        bins[pl.ds(L, L)] = zeros

        UB = 8

        def count_body(i, _, srck=srck, shift=shift):
            ks = [srck[pl.ds((i * UB + u) * L, L)] for u in range(UB)]
            dds = [31 - ((k >> shift) & 31) for k in ks]
            for dd in dds:
                plsc.addupdate_scatter(bins, [dd], ones)
            return 0

        lax.fori_loop(0, nv_sel // UB, count_body, 0)

        v0 = bins[pl.ds(0, L)]
        v1 = bins[pl.ds(L, L)]
        bins[pl.ds(0, L)] = plsc.cumsum(v0) - v0
        bins[pl.ds(L, L)] = plsc.cumsum(v1) - v1 + jnp.sum(v0)

        UP = 4

        def perm_body(i, _, srck=srck, srci=srci, dstk=dstk, dsti=dsti,
                      shift=shift):
            ks = [srck[pl.ds((i * UP + u) * L, L)] for u in range(UP)]
            ivs = [srci[pl.ds((i * UP + u) * L, L)] for u in range(UP)]
            dds = [31 - ((k >> shift) & 31) for k in ks]
            scans = [plsc.scan_count(dd) for dd in dds]
            for u in range(UP):
                occ, lm = scans[u]
                base = plsc.load_gather(bins, [dds[u]])
                plsc.addupdate_scatter(bins, [dds[u]], occ, mask=lm)
                dest = base + occ - 1
                plsc.store_scatter(dstk, [dest], ks[u])
                plsc.store_scatter(dsti, [dest], ivs[u])
            return 0

        lax.fori_loop(0, nv_sel // UP, perm_body, 0)

    UO = 8

    def out_body(i, _):
        ks = [selk[pl.ds((i * UO + u) * L, L)] for u in range(UO)]
        vs = [plsc.bitcast(k, jnp.float32) for k in ks]
        for u in range(UO):
            outv[pl.ds((i * UO + u) * L, L)] = vs[u]
        return 0

    lax.fori_loop(0, TOPK // L // UO, out_body, 0)

    pltpu.sync_copy(outv, vals_hbm.at[pl.ds(e * TOPK, TOPK)])
    pltpu.sync_copy(seli.at[pl.ds(0, TOPK)], idx_hbm.at[pl.ds(e * TOPK, TOPK)])


def _make_topk_sc():
    mesh = plsc.VectorSubcoreMesh(core_axis_name="c", subcore_axis_name="s")

    @functools.partial(
        pl.kernel,
        out_type=(
            jax.ShapeDtypeStruct((NUM_EXPERTS * TOPK,), jnp.float32),
            jax.ShapeDtypeStruct((NUM_EXPERTS * TOPK,), jnp.int32),
        ),
        mesh=mesh,
        compiler_params=pltpu.CompilerParams(needs_layout_passes=False),
        scratch_types=[
            pltpu.VMEM((N_TOKENS,), jnp.int32),
            pltpu.VMEM((N_TOKENS,), jnp.int32),
            pltpu.VMEM((N_TOKENS,), jnp.int32),
            pltpu.VMEM((N_TOKENS // L,), jnp.int32),
            pltpu.VMEM((TOPK,), jnp.int32),
            pltpu.VMEM((TOPK,), jnp.int32),
            pltpu.VMEM((TOPK,), jnp.int32),
            pltpu.VMEM((TOPK,), jnp.int32),
            pltpu.VMEM((2 * L,), jnp.int32),
            pltpu.VMEM((TOPK,), jnp.float32),
            pltpu.SMEM((8,), jnp.int32),
            pltpu.SemaphoreType.DMA,
            pltpu.SemaphoreType.DMA,
        ],
    )
    def topk_sc(scores_hbm, vals_hbm, idx_hbm,
                keys, keys2, hist, coarse, selk, seli, selk2, seli2, bins,
                outv, smem, sem0, sem1):
        wid = lax.axis_index("s") * 2 + lax.axis_index("c")
        e0 = wid * 2
        cp0 = pltpu.async_copy(
            scores_hbm.at[pl.ds(e0 * N_TOKENS, N_TOKENS)], keys, sem0)
        cp1 = pltpu.async_copy(
            scores_hbm.at[pl.ds((e0 + 1) * N_TOKENS, N_TOKENS)], keys2, sem1)
        cp0.wait()
        _topk_row(vals_hbm, idx_hbm, e0,
                  keys, hist, coarse, selk, seli, selk2, seli2, bins,
                  outv, smem)
        cp1.wait()
        _topk_row(vals_hbm, idx_hbm, e0 + 1,
                  keys2, hist, coarse, selk, seli, selk2, seli2, bins,
                  outv, smem)

    return topk_sc


_topk_sc = _make_topk_sc()


@jax.jit
def kernel(x, W_gate, b_gate):
    score_bits = _gate_scores(x, W_gate, b_gate)
    vals, idx = _topk_sc(score_bits.reshape(-1))
    return vals.reshape(NUM_EXPERTS, TOPK), idx.reshape(NUM_EXPERTS, TOPK)


# speedup vs baseline: 1.2520x; 1.0099x over previous
import functools

import jax
import jax.numpy as jnp
from jax import lax
from jax.experimental import pallas as pl
from jax.experimental.pallas import tpu as pltpu
from jax.experimental.pallas import tpu_sc as plsc

DIM = 768
NUM_EXPERTS = 64
N_TOKENS = 32768
TOPK = 512
BT = 4096

L = 16
NV = N_TOKENS // L


def _gate_body(x_ref, w_ref, b_ref, out_ref):
    xb = x_ref[...]
    w = w_ref[...]
    logits = lax.dot_general(
        w, xb, (((1,), (1,)), ((), ())),
        preferred_element_type=jnp.float32)
    logits = logits + b_ref[...][:, None]
    scores = jax.nn.sigmoid(logits)
    bits = lax.bitcast_convert_type(scores, jnp.int32)
    out_ref[...] = bits.reshape(NUM_EXPERTS, BT // 1024, 8, 128)


def _gate_scores(x, W_gate, b_gate):
    grid = (N_TOKENS // BT,)
    return pl.pallas_call(
        _gate_body,
        grid=grid,
        in_specs=[
            pl.BlockSpec((BT, DIM), lambda i: (i, 0)),
            pl.BlockSpec((NUM_EXPERTS, DIM), lambda i: (0, 0)),
            pl.BlockSpec((NUM_EXPERTS,), lambda i: (0,)),
        ],
        out_specs=pl.BlockSpec(
            (NUM_EXPERTS, BT // 1024, 8, 128), lambda i: (0, i, 0, 0)),
        out_shape=jax.ShapeDtypeStruct(
            (NUM_EXPERTS, N_TOKENS // 1024, 8, 128), jnp.int32),
    )(x, W_gate, b_gate)


def _iota16():
    return lax.broadcasted_iota(jnp.int32, (L,), 0)


def _lane_cross(v, carry, target, iota):
    rv = lax.rev(v, (0,))
    dcum = lax.rev(plsc.cumsum(rv), (0,)) + carry
    cond_v = (dcum >= target).astype(jnp.int32)
    lane = jnp.sum(cond_v) - 1
    sel = iota == lane
    zeros = jnp.zeros((L,), jnp.int32)
    above = jnp.sum(jnp.where(sel, dcum - v, zeros))
    return lane, above


def _find_threshold(hist, coarse, ncoarse_v, target, smem, slot):
    iota = _iota16()

    def cond(state):
        _, carry = state
        return carry < target

    def body(state):
        j, carry = state
        v = coarse[pl.ds(j * L, L)]
        s = jnp.sum(v)
        new = carry + s

        @pl.when(new >= target)
        def _():
            lane, above = _lane_cross(v, carry, target, iota)
            smem[6] = j * L + lane
            smem[7] = above

        return j - 1, new

    lax.while_loop(cond, body, (jnp.int32(ncoarse_v - 1), jnp.int32(0)))

    cb = smem[6]
    carry2 = smem[7]
    v = hist[pl.ds(cb * L, L)]
    lane, above = _lane_cross(v, carry2, target, iota)
    smem[slot] = cb * L + lane
    smem[slot + 1] = above


def _topk_row(vals_hbm, idx_hbm, e,
              keys, hist, coarse, selk, seli, selk2, seli2, bins, outv, smem):
    iota = _iota16()
    zeros = jnp.zeros((L,), jnp.int32)
    ones = jnp.ones((L,), jnp.int32)

    U = 8
    NCV = NV // L

    def clear_body(i, _):
        for u in range(U):
            hist[pl.ds((i * U + u) * L, L)] = zeros
        return 0

    def clear_coarse_body(i, _):
        for u in range(U):
            coarse[pl.ds((i * U + u) * L, L)] = zeros
        return 0

    lax.fori_loop(0, NV // U, clear_body, 0)
    lax.fori_loop(0, NCV // U, clear_coarse_body, 0)

    def hist_hi_body(i, _):
        ks = [keys[pl.ds((i * U + u) * L, L)] for u in range(U)]
        bs = [k >> 15 for k in ks]
        cbs = [k >> 19 for k in ks]
        for b, cb in zip(bs, cbs):
            plsc.addupdate_scatter(hist, [b], ones)
            plsc.addupdate_scatter(coarse, [cb], ones)
        return 0

    lax.fori_loop(0, NV // U, hist_hi_body, 0)

    _find_threshold(hist, coarse, NCV, jnp.int32(TOPK), smem, 0)
    h_star = smem[0]
    c_gt = smem[1]

    lax.fori_loop(0, NV // U, clear_body, 0)
    lax.fori_loop(0, NCV // U, clear_coarse_body, 0)

    def hist_lo_body(i, _):
        ks = [keys[pl.ds((i * U + u) * L, L)] for u in range(U)]
        els = [(k >> 15) == h_star for k in ks]
        lows = [k & 0x7FFF for k in ks]
        for lo, el in zip(lows, els):
            plsc.addupdate_scatter(hist, [lo], ones, mask=el)
            plsc.addupdate_scatter(coarse, [lo >> 4], ones, mask=el)
        return 0

    lax.fori_loop(0, NV // U, hist_lo_body, 0)

    _find_threshold(hist, coarse, NCV, TOPK - c_gt, smem, 3)
    l_star = smem[3]
    c_gt2 = smem[4]

    t_key = (h_star << 15) | l_star
    c_sel = c_gt + c_gt2

    UC = 8
    t_vec = zeros + t_key

    def collect_body(i, carry):
        offg, offe, idxv = carry
        ks = [keys[pl.ds((i * UC + u) * L, L)] for u in range(UC)]
        gts = [k > t_key for k in ks]
        eqs = [k == t_key for k in ks]
        prefs_g = [plsc.cumsum(gt.astype(jnp.int32)) for gt in gts]
        prefs_e = [plsc.cumsum(eq.astype(jnp.int32)) for eq in eqs]
        cnts_g = [plsc.all_reduce_population_count(gt) for gt in gts]
        cnts_e = [plsc.all_reduce_population_count(eq) for eq in eqs]
        for u in range(UC):
            dest_g = offg + prefs_g[u] - 1
            plsc.store_scatter(selk, [dest_g], ks[u], mask=gts[u])
            plsc.store_scatter(seli, [dest_g], idxv + u * L, mask=gts[u])
            offg = offg + cnts_g[u]
        for u in range(UC):
            dest_e = offe + prefs_e[u] - 1
            okm = jnp.logical_and(eqs[u], dest_e < TOPK)
            plsc.store_scatter(selk, [dest_e], t_vec, mask=okm)
            plsc.store_scatter(seli, [dest_e], idxv + u * L, mask=okm)
            offe = offe + cnts_e[u]
        return offg, offe, idxv + UC * L

    lax.fori_loop(0, NV // UC, collect_body, (zeros, zeros + c_sel, iota))

    nv_sel = TOPK // L
    bufs = [(selk, seli), (selk2, seli2)]
    for p in range(5):
        srck, srci = bufs[p % 2]
        dstk, dsti = bufs[(p + 1) % 2]
        shift = 6 * p

        for q in range(4):
            bins[pl.ds(q * L, L)] = zeros

        UB = 8

        def count_body(i, _, srck=srck, shift=shift):
            ks = [srck[pl.ds((i * UB + u) * L, L)] for u in range(UB)]
            dds = [63 - ((k >> shift) & 63) for k in ks]
            for dd in dds:
                plsc.addupdate_scatter(bins, [dd], ones)
            return 0

        lax.fori_loop(0, nv_sel // UB, count_body, 0)

        vs_b = [bins[pl.ds(q * L, L)] for q in range(4)]
        carry = jnp.int32(0)
        for q in range(4):
            bins[pl.ds(q * L, L)] = plsc.cumsum(vs_b[q]) - vs_b[q] + carry
            carry = carry + jnp.sum(vs_b[q])

        UP = 4

        def perm_body(i, _, srck=srck, srci=srci, dstk=dstk, dsti=dsti,
                      shift=shift):
            ks = [srck[pl.ds((i * UP + u) * L, L)] for u in range(UP)]
            ivs = [srci[pl.ds((i * UP + u) * L, L)] for u in range(UP)]
            dds = [63 - ((k >> shift) & 63) for k in ks]
            scans = [plsc.scan_count(dd) for dd in dds]
            for u in range(UP):
                occ, lm = scans[u]
                base = plsc.load_gather(bins, [dds[u]])
                plsc.addupdate_scatter(bins, [dds[u]], occ, mask=lm)
                dest = base + occ - 1
                plsc.store_scatter(dstk, [dest], ks[u])
                plsc.store_scatter(dsti, [dest], ivs[u])
            return 0

        lax.fori_loop(0, nv_sel // UP, perm_body, 0)

    UO = 8

    def out_body(i, _):
        ks = [selk2[pl.ds((i * UO + u) * L, L)] for u in range(UO)]
        vs = [plsc.bitcast(k, jnp.float32) for k in ks]
        for u in range(UO):
            outv[pl.ds((i * UO + u) * L, L)] = vs[u]
        return 0

    lax.fori_loop(0, TOPK // L // UO, out_body, 0)

    pltpu.sync_copy(outv, vals_hbm.at[pl.ds(e * TOPK, TOPK)])
    pltpu.sync_copy(seli2.at[pl.ds(0, TOPK)],
                    idx_hbm.at[pl.ds(e * TOPK, TOPK)])


def _make_topk_sc():
    mesh = plsc.VectorSubcoreMesh(core_axis_name="c", subcore_axis_name="s")

    @functools.partial(
        pl.kernel,
        out_type=(
            jax.ShapeDtypeStruct((NUM_EXPERTS * TOPK,), jnp.float32),
            jax.ShapeDtypeStruct((NUM_EXPERTS * TOPK,), jnp.int32),
        ),
        mesh=mesh,
        compiler_params=pltpu.CompilerParams(needs_layout_passes=False),
        scratch_types=[
            pltpu.VMEM((N_TOKENS,), jnp.int32),
            pltpu.VMEM((N_TOKENS,), jnp.int32),
            pltpu.VMEM((N_TOKENS,), jnp.int32),
            pltpu.VMEM((N_TOKENS // L,), jnp.int32),
            pltpu.VMEM((TOPK,), jnp.int32),
            pltpu.VMEM((TOPK,), jnp.int32),
            pltpu.VMEM((TOPK,), jnp.int32),
            pltpu.VMEM((TOPK,), jnp.int32),
            pltpu.VMEM((4 * L,), jnp.int32),
            pltpu.VMEM((TOPK,), jnp.float32),
            pltpu.SMEM((8,), jnp.int32),
            pltpu.SemaphoreType.DMA,
            pltpu.SemaphoreType.DMA,
        ],
    )
    def topk_sc(scores_hbm, vals_hbm, idx_hbm,
                keys, keys2, hist, coarse, selk, seli, selk2, seli2, bins,
                outv, smem, sem0, sem1):
        wid = lax.axis_index("s") * 2 + lax.axis_index("c")
        e0 = wid * 2
        cp0 = pltpu.async_copy(
            scores_hbm.at[pl.ds(e0 * N_TOKENS, N_TOKENS)], keys, sem0)
        cp1 = pltpu.async_copy(
            scores_hbm.at[pl.ds((e0 + 1) * N_TOKENS, N_TOKENS)], keys2, sem1)
        cp0.wait()
        _topk_row(vals_hbm, idx_hbm, e0,
                  keys, hist, coarse, selk, seli, selk2, seli2, bins,
                  outv, smem)
        cp1.wait()
        _topk_row(vals_hbm, idx_hbm, e0 + 1,
                  keys2, hist, coarse, selk, seli, selk2, seli2, bins,
                  outv, smem)

    return topk_sc


_topk_sc = _make_topk_sc()


@jax.jit
def kernel(x, W_gate, b_gate):
    score_bits = _gate_scores(x, W_gate, b_gate)
    vals, idx = _topk_sc(score_bits.reshape(-1))
    return vals.reshape(NUM_EXPERTS, TOPK), idx.reshape(NUM_EXPERTS, TOPK)
